# baseline (speedup 1.0000x reference)
import functools

import jax
import jax.numpy as jnp
from jax.experimental import pallas as pl
from jax.experimental.pallas import tpu as pltpu


def _rvq_body(x_ref, cb_hbm, quants_ref, codes_ref, loss_ref,
              cb_hi, cbnh_ref, stage_ref, sem, *, depth, k, d, tt, kc):
    g = pl.program_id(0)
    nk = k // kc

    @pl.when(g == 0)
    def _init():
        ones_row = jnp.ones((1, d), dtype=jnp.float32)
        for i in range(depth):
            for c in range(nk):
                cp = pltpu.make_async_copy(
                    cb_hbm.at[i, pl.ds(c * kc, kc), :], stage_ref, sem)
                cp.start()
                cp.wait()
                cb_c = stage_ref[...]
                cb_hi[i, c * kc:(c + 1) * kc, :] = cb_c.astype(jnp.bfloat16)
                cbnh_ref[i:i + 1, c * kc:(c + 1) * kc] = jax.lax.dot_general(
                    ones_row, (0.5 * cb_c) * cb_c, (((1,), (1,)), ((), ())),
                    preferred_element_type=jnp.float32,
                    precision=jax.lax.Precision.HIGHEST)
        loss_ref[...] = jnp.zeros_like(loss_ref)

    x = x_ref[...]
    r = x
    idx_cols = []
    loss_acc = jnp.zeros_like(loss_ref)
    row_iota = jax.lax.broadcasted_iota(jnp.int32, loss_ref.shape, 0)
    lane = jax.lax.broadcasted_iota(jnp.int32, (tt, k), 1)
    for i in range(depth):
        rb = r.astype(jnp.bfloat16)
        s = jax.lax.dot_general(
            rb, cb_hi[i], (((1,), (1,)), ((), ())),
            preferred_element_type=jnp.float32)
        dist = cbnh_ref[i:i + 1, :] - s
        cmin = jnp.min(dist, axis=1, keepdims=True)
        best_idx = jnp.min(jnp.where(dist == cmin, lane, k),
                           axis=1, keepdims=True)
        oh = (lane == best_idx).astype(jnp.bfloat16)
        quant = jax.lax.dot_general(
            oh, cb_hi[i], (((1,), (0,)), ((), ())),
            preferred_element_type=jnp.float32)
        r = r - quant
        ssq = jnp.sum(r * r)
        loss_acc = loss_acc + jnp.where(row_iota == i, ssq, 0.0)
        idx_cols.append(best_idx)

    quants_ref[...] = x - r
    codes_ref[...] = jnp.concatenate(idx_cols, axis=1)
    loss_ref[...] += loss_acc


def kernel(x, codebooks):
    b, t, d = x.shape
    depth, k, _ = codebooks.shape
    n = b * t
    tt = 256 if n % 256 == 0 else n
    kc = 2048 if k % 2048 == 0 else k
    x_flat = x.reshape(n, d)

    body = functools.partial(_rvq_body, depth=depth, k=k, d=d, tt=tt, kc=kc)
    quants_flat, codes_flat, loss_part = pl.pallas_call(
        body,
        grid=(n // tt,),
        in_specs=[
            pl.BlockSpec((tt, d), lambda g: (g, 0)),
            pl.BlockSpec(memory_space=pl.ANY),
        ],
        out_specs=[
            pl.BlockSpec((tt, d), lambda g: (g, 0)),
            pl.BlockSpec((tt, depth), lambda g: (g, 0)),
            pl.BlockSpec((8, 128), lambda g: (0, 0)),
        ],
        out_shape=[
            jax.ShapeDtypeStruct((n, d), jnp.float32),
            jax.ShapeDtypeStruct((n, depth), jnp.int32),
            jax.ShapeDtypeStruct((8, 128), jnp.float32),
        ],
        scratch_shapes=[
            pltpu.VMEM((depth, k, d), jnp.bfloat16),
            pltpu.VMEM((8, k), jnp.float32),
            pltpu.VMEM((2048 if k % 2048 == 0 else k, d), jnp.float32),
            pltpu.SemaphoreType.DMA,
        ],
        compiler_params=pltpu.CompilerParams(
            dimension_semantics=("arbitrary",)),
    )(x_flat, codebooks)

    quants = quants_flat.reshape(b, t, d)
    codes = codes_flat.reshape(b, t, depth)
    loss = jnp.sum(loss_part[:depth, 0]) / (depth * n * d)
    return quants, loss, codes


# speedup vs baseline: 1.3692x; 1.3692x over previous
import functools

import jax
import jax.numpy as jnp
from jax.experimental import pallas as pl
from jax.experimental.pallas import tpu as pltpu


def _rvq_body(x_ref, cb_hbm, quants_ref, codes_ref, loss_ref,
              cb_hi, cbnh_ref, stage_ref, sem, *, depth, k, d, tt, kc):
    g = pl.program_id(0)
    nk = k // kc

    @pl.when(g == 0)
    def _init():
        ones_row = jnp.ones((1, d), dtype=jnp.float32)
        for i in range(depth):
            for c in range(nk):
                cp = pltpu.make_async_copy(
                    cb_hbm.at[i, pl.ds(c * kc, kc), :], stage_ref, sem)
                cp.start()
                cp.wait()
                cb_c = stage_ref[...]
                cb_hi[i, c * kc:(c + 1) * kc, :] = cb_c.astype(jnp.bfloat16)
                cbnh_ref[i:i + 1, c * kc:(c + 1) * kc] = jax.lax.dot_general(
                    ones_row, (0.5 * cb_c) * cb_c, (((1,), (1,)), ((), ())),
                    preferred_element_type=jnp.float32,
                    precision=jax.lax.Precision.HIGHEST)
        loss_ref[...] = jnp.zeros_like(loss_ref)

    x = x_ref[...]
    r = x
    idx_cols = []
    loss_acc = jnp.zeros_like(loss_ref)
    row_iota = jax.lax.broadcasted_iota(jnp.int32, loss_ref.shape, 0)
    lane = jax.lax.broadcasted_iota(jnp.int32, (tt, kc), 1)
    for i in range(depth):
        rb = r.astype(jnp.bfloat16)
        best_val = jnp.full((tt, 1), jnp.inf, dtype=jnp.float32)
        best_idx = jnp.zeros((tt, 1), dtype=jnp.int32)
        for c in range(nk):
            hi_c = cb_hi[i, c * kc:(c + 1) * kc, :]
            s = jax.lax.dot_general(
                rb, hi_c, (((1,), (1,)), ((), ())),
                preferred_element_type=jnp.float32)
            dist = cbnh_ref[i:i + 1, c * kc:(c + 1) * kc] - s
            cmin = jnp.min(dist, axis=1, keepdims=True)
            cidx = jnp.min(jnp.where(dist == cmin, lane, kc),
                           axis=1, keepdims=True) + c * kc
            upd = cmin < best_val
            best_val = jnp.where(upd, cmin, best_val)
            best_idx = jnp.where(upd, cidx, best_idx)
        quant = jnp.zeros((tt, d), dtype=jnp.float32)
        for c in range(nk):
            oh = (lane == best_idx - c * kc).astype(jnp.bfloat16)
            quant = quant + jax.lax.dot_general(
                oh, cb_hi[i, c * kc:(c + 1) * kc, :],
                (((1,), (0,)), ((), ())), preferred_element_type=jnp.float32)
        r = r - quant
        ssq = jnp.sum(r * r)
        loss_acc = loss_acc + jnp.where(row_iota == i, ssq, 0.0)
        idx_cols.append(best_idx)

    quants_ref[...] = x - r
    codes_ref[...] = jnp.concatenate(idx_cols, axis=1)
    loss_ref[...] += loss_acc


def kernel(x, codebooks):
    b, t, d = x.shape
    depth, k, _ = codebooks.shape
    n = b * t
    tt = 512 if n % 512 == 0 else n
    kc = 2048 if k % 2048 == 0 else k
    x_flat = x.reshape(n, d)

    body = functools.partial(_rvq_body, depth=depth, k=k, d=d, tt=tt, kc=kc)
    quants_flat, codes_flat, loss_part = pl.pallas_call(
        body,
        grid=(n // tt,),
        in_specs=[
            pl.BlockSpec((tt, d), lambda g: (g, 0)),
            pl.BlockSpec(memory_space=pl.ANY),
        ],
        out_specs=[
            pl.BlockSpec((tt, d), lambda g: (g, 0)),
            pl.BlockSpec((tt, depth), lambda g: (g, 0)),
            pl.BlockSpec((8, 128), lambda g: (0, 0)),
        ],
        out_shape=[
            jax.ShapeDtypeStruct((n, d), jnp.float32),
            jax.ShapeDtypeStruct((n, depth), jnp.int32),
            jax.ShapeDtypeStruct((8, 128), jnp.float32),
        ],
        scratch_shapes=[
            pltpu.VMEM((depth, k, d), jnp.bfloat16),
            pltpu.VMEM((8, k), jnp.float32),
            pltpu.VMEM((2048 if k % 2048 == 0 else k, d), jnp.float32),
            pltpu.SemaphoreType.DMA,
        ],
        compiler_params=pltpu.CompilerParams(
            dimension_semantics=("arbitrary",)),
    )(x_flat, codebooks)

    quants = quants_flat.reshape(b, t, d)
    codes = codes_flat.reshape(b, t, depth)
    loss = jnp.sum(loss_part[:depth, 0]) / (depth * n * d)
    return quants, loss, codes
